# baseline (speedup 1.0000x reference)
import functools

import jax
import jax.numpy as jnp
import numpy as np
from jax import lax
from jax.experimental import pallas as pl
from jax.experimental.pallas import tpu as pltpu
from jax.experimental.pallas import tpu_sc as plsc

_PRED_FREQ = np.array([712432, 253342, 208287, 197550, 66425, 47342, 33637,
                       32347, 21575, 15457, 13715, 13360, 10191, 9903, 9894,
                       9317, 9145, 8856, 6712, 5213, 4688, 4613, 4507, 4465,
                       4021, 3810, 3806, 3739, 3624, 3490, 3477, 3411, 3288,
                       3095, 3092, 3083, 2945, 2721, 2517, 2450, 2312, 2253,
                       2241, 2065, 1829, 1603, 1413, 1225, 793, 712, 663],
                      dtype=np.float32)
_ALPHA = 0.4
_FG_BOOST = 1.2
_N = 262144
_C = 51

_BN = 8192
_Q = 65536
_NW = 32
_RW = _Q // _NW
_RC = 512
_WBG = float(_ALPHA)
_WFG = float((1.0 - _ALPHA) * _FG_BOOST)
_NEG = -1e30


def _log_priors() -> np.ndarray:
    pf = np.nan_to_num(_PRED_FREQ, nan=1e-06)
    pf = np.clip(pf, 1e-12, None)
    priors = pf / (pf.sum(dtype=np.float32) + 1e-12)
    return np.clip(np.log(priors + 1e-12), -20.0, 20.0).astype(np.float32)


_LP = _log_priors()


def _tc_body(x_ref, tp_ref, lp_ref, loss_ref, wsum_ref):
    i = pl.program_id(0)
    xt = x_ref[...].T
    tp = tp_ref[0]
    lp = lp_ref[...]
    adj = jnp.clip(xt + lp, -50.0, 50.0)
    e = jnp.exp(adj)
    cls = jax.lax.broadcasted_iota(jnp.int32, (_C, _BN), 0)
    ev = jnp.sum(jnp.where(cls == tp, e, 0.0), axis=0, keepdims=True)
    s = jnp.sum(e, axis=0, keepdims=True)
    ce = jnp.log(s) - jnp.log(ev)
    pt = ev / s
    fw = (1.0 - pt) * (1.0 - pt)
    w = jnp.where(tp == 0, jnp.float32(_WBG), jnp.float32(_WFG))
    part_loss = jnp.sum(ce * fw * w)
    part_w = jnp.sum(w)

    @pl.when(i == 0)
    def _():
        loss_ref[0, 0] = 0.0
        wsum_ref[0, 0] = 0.0

    loss_ref[0, 0] += part_loss
    wsum_ref[0, 0] += part_w


def _sc_body(x_hbm, t_hbm, lp_hbm, s_hbm, ev_hbm, xv, tv_i, lpv, sv, evv):
    wid = lax.axis_index("s") * 2 + lax.axis_index("c")
    row0 = _N - _Q
    pltpu.sync_copy(lp_hbm, lpv)
    offs = (0, 16, 32, 35)
    lps = [lpv[pl.ds(o, 16)] for o in offs]
    iota16 = lax.broadcasted_iota(jnp.int32, (16,), 0)
    iotas = [iota16 + o for o in offs]
    tailmask = iota16 >= 13

    for chunk in range(_RW // _RC):
        base = row0 + wid * _RW + chunk * _RC
        pltpu.sync_copy(x_hbm.at[pl.ds(base, _RC), :], xv)
        pltpu.sync_copy(t_hbm.at[pl.ds(base, _RC)], tv_i)

        def group(g, carry):
            tv16 = tv_i[pl.ds(16 * g, 16)]
            for j in range(16):
                r = 16 * g + j
                tsc = tv16[j]
                s_acc = jnp.zeros((16,), jnp.float32)
                ev_acc = jnp.zeros((16,), jnp.float32)
                for k in range(4):
                    v = xv[r, pl.ds(offs[k], 16)] + lps[k]
                    if k == 3:
                        v = jnp.where(tailmask, v, jnp.float32(_NEG))
                    e = jnp.exp(v)
                    s_acc = s_acc + e
                    ev_acc = ev_acc + jnp.where(iotas[k] == tsc, e, 0.0)
                sv[pl.ds(16 * r, 16)] = s_acc
                evv[pl.ds(16 * r, 16)] = ev_acc
            return carry

        lax.fori_loop(0, _RC // 16, group, 0)
        obase = (wid * _RW + chunk * _RC) * 16
        pltpu.sync_copy(sv, s_hbm.at[pl.ds(obase, _RC * 16)])
        pltpu.sync_copy(evv, ev_hbm.at[pl.ds(obase, _RC * 16)])


def _sc_stats(logits, t32, lp64):
    mesh = plsc.VectorSubcoreMesh(core_axis_name="c", subcore_axis_name="s")
    f = functools.partial(
        pl.kernel, mesh=mesh,
        out_type=[jax.ShapeDtypeStruct((_Q * 16,), jnp.float32),
                  jax.ShapeDtypeStruct((_Q * 16,), jnp.float32)],
        scratch_types=[pltpu.VMEM((_RC, _C), jnp.float32),
                       pltpu.VMEM((_RC,), jnp.int32),
                       pltpu.VMEM((64,), jnp.float32),
                       pltpu.VMEM((_RC * 16,), jnp.float32),
                       pltpu.VMEM((_RC * 16,), jnp.float32)],
        compiler_params=pltpu.CompilerParams(use_tc_tiling_on_sc=True),
    )(_sc_body)
    return f(logits, t32, lp64)


_BQ = 8192


def _merge_body(s_ref, ev_ref, tq_ref, loss_ref, wsum_ref):
    i = pl.program_id(0)
    st = s_ref[...].T
    et = ev_ref[...].T
    tp = tq_ref[0]
    s = jnp.sum(st, axis=0, keepdims=True)
    ev = jnp.sum(et, axis=0, keepdims=True)
    ce = jnp.log(s) - jnp.log(ev)
    pt = ev / s
    fw = (1.0 - pt) * (1.0 - pt)
    w = jnp.where(tp == 0, jnp.float32(_WBG), jnp.float32(_WFG))
    part_loss = jnp.sum(ce * fw * w)
    part_w = jnp.sum(w)

    @pl.when(i == 0)
    def _():
        loss_ref[0, 0] = 0.0
        wsum_ref[0, 0] = 0.0

    loss_ref[0, 0] += part_loss
    wsum_ref[0, 0] += part_w


def kernel(logits, target):
    logits = logits.astype(jnp.float32)
    t32 = target.astype(jnp.int32)
    lp = jnp.asarray(_LP).reshape(_C, 1)
    lp64 = jnp.concatenate([jnp.asarray(_LP), jnp.zeros(13, jnp.float32)])

    nt = _N - _Q
    s_q, ev_q = _sc_stats(logits, t32, lp64)

    t_pack = t32[:nt].reshape(nt // _BN, 1, _BN)
    grid = nt // _BN
    loss_a, wsum_a = pl.pallas_call(
        _tc_body,
        grid=(grid,),
        in_specs=[
            pl.BlockSpec((_BN, _C), lambda i: (i, 0)),
            pl.BlockSpec((1, 1, _BN), lambda i: (i, 0, 0)),
            pl.BlockSpec((_C, 1), lambda i: (0, 0)),
        ],
        out_specs=[
            pl.BlockSpec((1, 1), lambda i: (0, 0), memory_space=pltpu.SMEM),
            pl.BlockSpec((1, 1), lambda i: (0, 0), memory_space=pltpu.SMEM),
        ],
        out_shape=[
            jax.ShapeDtypeStruct((1, 1), jnp.float32),
            jax.ShapeDtypeStruct((1, 1), jnp.float32),
        ],
    )(logits, t_pack, lp)

    mgrid = _Q // _BQ
    tq_pack = t32[nt:].reshape(mgrid, 1, _BQ)
    loss_b, wsum_b = pl.pallas_call(
        _merge_body,
        grid=(mgrid,),
        in_specs=[
            pl.BlockSpec((_BQ, 16), lambda i: (i, 0)),
            pl.BlockSpec((_BQ, 16), lambda i: (i, 0)),
            pl.BlockSpec((1, 1, _BQ), lambda i: (i, 0, 0)),
        ],
        out_specs=[
            pl.BlockSpec((1, 1), lambda i: (0, 0), memory_space=pltpu.SMEM),
            pl.BlockSpec((1, 1), lambda i: (0, 0), memory_space=pltpu.SMEM),
        ],
        out_shape=[
            jax.ShapeDtypeStruct((1, 1), jnp.float32),
            jax.ShapeDtypeStruct((1, 1), jnp.float32),
        ],
    )(s_q.reshape(_Q, 16), ev_q.reshape(_Q, 16), tq_pack)

    loss = loss_a[0, 0] + loss_b[0, 0]
    wsum = wsum_a[0, 0] + wsum_b[0, 0]
    return loss / jnp.clip(wsum, 1.0, None)


# speedup vs baseline: 5.2977x; 5.2977x over previous
import jax
import jax.numpy as jnp
import numpy as np
from jax.experimental import pallas as pl
from jax.experimental.pallas import tpu as pltpu

_PRED_FREQ = np.array([712432, 253342, 208287, 197550, 66425, 47342, 33637,
                       32347, 21575, 15457, 13715, 13360, 10191, 9903, 9894,
                       9317, 9145, 8856, 6712, 5213, 4688, 4613, 4507, 4465,
                       4021, 3810, 3806, 3739, 3624, 3490, 3477, 3411, 3288,
                       3095, 3092, 3083, 2945, 2721, 2517, 2450, 2312, 2253,
                       2241, 2065, 1829, 1603, 1413, 1225, 793, 712, 663],
                      dtype=np.float32)
_GAMMA = 2.0
_ALPHA = 0.4
_TAU = 1.0
_FG_BOOST = 1.2
_N = 262144
_C = 51

_BN = 8192


def _log_priors() -> np.ndarray:
    pf = np.nan_to_num(_PRED_FREQ, nan=1e-06)
    pf = np.clip(pf, 1e-12, None)
    priors = pf / (pf.sum(dtype=np.float32) + 1e-12)
    return np.clip(np.log(priors + 1e-12), -20.0, 20.0).astype(np.float32)


def _body(x_ref, tp_ref, lp_ref, loss_ref, wsum_ref):
    i = pl.program_id(0)
    xt = x_ref[...]
    tp = tp_ref[0]
    lp = lp_ref[...]
    adj = jnp.clip(xt + lp, -50.0, 50.0)
    e = jnp.exp(adj)
    cls = jax.lax.broadcasted_iota(jnp.int32, (_C, _BN), 0)
    ev = jnp.sum(jnp.where(cls == tp, e, 0.0), axis=0, keepdims=True)
    s = jnp.sum(e, axis=0, keepdims=True)
    ce = jnp.log(s) - jnp.log(ev)
    pt = ev / s
    fw = (1.0 - pt) * (1.0 - pt)
    w = jnp.where(tp == 0, jnp.float32(_ALPHA),
                  jnp.float32((1.0 - _ALPHA) * _FG_BOOST))
    part_loss = jnp.sum(ce * fw * w)
    part_w = jnp.sum(w)

    @pl.when(i == 0)
    def _():
        loss_ref[0, 0] = 0.0
        wsum_ref[0, 0] = 0.0

    loss_ref[0, 0] += part_loss
    wsum_ref[0, 0] += part_w


def kernel(logits, target):
    xt = jnp.transpose(logits.astype(jnp.float32))
    t_pack = target.astype(jnp.int32).reshape(_N // _BN, 1, _BN)
    lp = jnp.asarray(_log_priors()).reshape(_C, 1)
    grid = _N // _BN
    loss_sum, w_sum = pl.pallas_call(
        _body,
        grid=(grid,),
        in_specs=[
            pl.BlockSpec((_C, _BN), lambda i: (0, i)),
            pl.BlockSpec((1, 1, _BN), lambda i: (i, 0, 0)),
            pl.BlockSpec((_C, 1), lambda i: (0, 0)),
        ],
        out_specs=[
            pl.BlockSpec((1, 1), lambda i: (0, 0), memory_space=pltpu.SMEM),
            pl.BlockSpec((1, 1), lambda i: (0, 0), memory_space=pltpu.SMEM),
        ],
        out_shape=[
            jax.ShapeDtypeStruct((1, 1), jnp.float32),
            jax.ShapeDtypeStruct((1, 1), jnp.float32),
        ],
    )(xt, t_pack, lp)
    normalizer = jnp.clip(w_sum[0, 0], 1.0, None)
    return loss_sum[0, 0] / normalizer


# speedup vs baseline: 6.7508x; 1.2743x over previous
import jax
import jax.numpy as jnp
import numpy as np
from jax.experimental import pallas as pl
from jax.experimental.pallas import tpu as pltpu

_PRED_FREQ = np.array([712432, 253342, 208287, 197550, 66425, 47342, 33637,
                       32347, 21575, 15457, 13715, 13360, 10191, 9903, 9894,
                       9317, 9145, 8856, 6712, 5213, 4688, 4613, 4507, 4465,
                       4021, 3810, 3806, 3739, 3624, 3490, 3477, 3411, 3288,
                       3095, 3092, 3083, 2945, 2721, 2517, 2450, 2312, 2253,
                       2241, 2065, 1829, 1603, 1413, 1225, 793, 712, 663],
                      dtype=np.float32)
_GAMMA = 2.0
_ALPHA = 0.4
_TAU = 1.0
_FG_BOOST = 1.2
_N = 262144
_C = 51

_BN = 16384


def _log_priors() -> np.ndarray:
    pf = np.nan_to_num(_PRED_FREQ, nan=1e-06)
    pf = np.clip(pf, 1e-12, None)
    priors = pf / (pf.sum(dtype=np.float32) + 1e-12)
    return np.clip(np.log(priors + 1e-12), -20.0, 20.0).astype(np.float32)


def _body(x_ref, tp_ref, lp_ref, loss_ref, wsum_ref):
    i = pl.program_id(0)
    xt = x_ref[...]
    tp = tp_ref[0]
    lp = lp_ref[...]
    adj = jnp.clip(xt + lp, -50.0, 50.0)
    e = jnp.exp(adj)
    cls = jax.lax.broadcasted_iota(jnp.int32, (_C, _BN), 0)
    ev = jnp.sum(jnp.where(cls == tp, e, 0.0), axis=0, keepdims=True)
    s = jnp.sum(e, axis=0, keepdims=True)
    ce = jnp.log(s) - jnp.log(ev)
    pt = ev / s
    fw = (1.0 - pt) * (1.0 - pt)
    w = jnp.where(tp == 0, jnp.float32(_ALPHA),
                  jnp.float32((1.0 - _ALPHA) * _FG_BOOST))
    part_loss = jnp.sum(ce * fw * w)
    part_w = jnp.sum(w)

    @pl.when(i == 0)
    def _():
        loss_ref[0, 0] = 0.0
        wsum_ref[0, 0] = 0.0

    loss_ref[0, 0] += part_loss
    wsum_ref[0, 0] += part_w


def kernel(logits, target):
    xt = jnp.transpose(logits.astype(jnp.float32))
    t_pack = target.astype(jnp.int32).reshape(_N // _BN, 1, _BN)
    lp = jnp.asarray(_log_priors()).reshape(_C, 1)
    grid = _N // _BN
    loss_sum, w_sum = pl.pallas_call(
        _body,
        grid=(grid,),
        in_specs=[
            pl.BlockSpec((_C, _BN), lambda i: (0, i)),
            pl.BlockSpec((1, 1, _BN), lambda i: (i, 0, 0)),
            pl.BlockSpec((_C, 1), lambda i: (0, 0)),
        ],
        out_specs=[
            pl.BlockSpec((1, 1), lambda i: (0, 0), memory_space=pltpu.SMEM),
            pl.BlockSpec((1, 1), lambda i: (0, 0), memory_space=pltpu.SMEM),
        ],
        out_shape=[
            jax.ShapeDtypeStruct((1, 1), jnp.float32),
            jax.ShapeDtypeStruct((1, 1), jnp.float32),
        ],
    )(xt, t_pack, lp)
    normalizer = jnp.clip(w_sum[0, 0], 1.0, None)
    return loss_sum[0, 0] / normalizer


# speedup vs baseline: 7.1648x; 1.0613x over previous
import jax
import jax.numpy as jnp
import numpy as np
from jax.experimental import pallas as pl
from jax.experimental.pallas import tpu as pltpu

_PRED_FREQ = np.array([712432, 253342, 208287, 197550, 66425, 47342, 33637,
                       32347, 21575, 15457, 13715, 13360, 10191, 9903, 9894,
                       9317, 9145, 8856, 6712, 5213, 4688, 4613, 4507, 4465,
                       4021, 3810, 3806, 3739, 3624, 3490, 3477, 3411, 3288,
                       3095, 3092, 3083, 2945, 2721, 2517, 2450, 2312, 2253,
                       2241, 2065, 1829, 1603, 1413, 1225, 793, 712, 663],
                      dtype=np.float32)
_GAMMA = 2.0
_ALPHA = 0.4
_TAU = 1.0
_FG_BOOST = 1.2
_N = 262144
_C = 51

_BN = 32768


def _log_priors() -> np.ndarray:
    pf = np.nan_to_num(_PRED_FREQ, nan=1e-06)
    pf = np.clip(pf, 1e-12, None)
    priors = pf / (pf.sum(dtype=np.float32) + 1e-12)
    return np.clip(np.log(priors + 1e-12), -20.0, 20.0).astype(np.float32)


def _body(x_ref, tp_ref, lp_ref, loss_ref, wsum_ref):
    i = pl.program_id(0)
    xt = x_ref[...]
    tp = tp_ref[0]
    lp = lp_ref[...]
    adj = jnp.clip(xt + lp, -50.0, 50.0)
    e = jnp.exp(adj)
    cls = jax.lax.broadcasted_iota(jnp.int32, (_C, _BN), 0)
    ev = jnp.sum(jnp.where(cls == tp, e, 0.0), axis=0, keepdims=True)
    s = jnp.sum(e, axis=0, keepdims=True)
    ce = jnp.log(s) - jnp.log(ev)
    pt = ev / s
    fw = (1.0 - pt) * (1.0 - pt)
    w = jnp.where(tp == 0, jnp.float32(_ALPHA),
                  jnp.float32((1.0 - _ALPHA) * _FG_BOOST))
    part_loss = jnp.sum(ce * fw * w)
    part_w = jnp.sum(w)

    @pl.when(i == 0)
    def _():
        loss_ref[0, 0] = 0.0
        wsum_ref[0, 0] = 0.0

    loss_ref[0, 0] += part_loss
    wsum_ref[0, 0] += part_w


def kernel(logits, target):
    xt = jnp.transpose(logits.astype(jnp.float32))
    t_pack = target.astype(jnp.int32).reshape(_N // _BN, 1, _BN)
    lp = jnp.asarray(_log_priors()).reshape(_C, 1)
    grid = _N // _BN
    loss_sum, w_sum = pl.pallas_call(
        _body,
        grid=(grid,),
        in_specs=[
            pl.BlockSpec((_C, _BN), lambda i: (0, i)),
            pl.BlockSpec((1, 1, _BN), lambda i: (i, 0, 0)),
            pl.BlockSpec((_C, 1), lambda i: (0, 0)),
        ],
        out_specs=[
            pl.BlockSpec((1, 1), lambda i: (0, 0), memory_space=pltpu.SMEM),
            pl.BlockSpec((1, 1), lambda i: (0, 0), memory_space=pltpu.SMEM),
        ],
        out_shape=[
            jax.ShapeDtypeStruct((1, 1), jnp.float32),
            jax.ShapeDtypeStruct((1, 1), jnp.float32),
        ],
    )(xt, t_pack, lp)
    normalizer = jnp.clip(w_sum[0, 0], 1.0, None)
    return loss_sum[0, 0] / normalizer
